# Initial kernel scaffold; baseline (speedup 1.0000x reference)
import jax
import jax.numpy as jnp
from jax.experimental import pallas as pl


def kernel(x, edge_index, batch, W_enc, b_enc, W1, b1, W2, b2):
    raise NotImplementedError("write your pallas kernel here")


# speedup vs baseline: 3.0681x; 3.0681x over previous
import functools

import jax
import jax.numpy as jnp
from jax import lax
from jax.experimental import pallas as pl
from jax.experimental.pallas import tpu as pltpu
from jax.experimental.pallas import tpu_sc as plsc

_N, _E, _D, _H, _C, _G = 10000, 320000, 128, 256, 10, 64
_NCORE, _NSUB = 2, 16
_NW = _NCORE * _NSUB
_K = 128
_CHUNKS = 80
_EPW = _K * _CHUNKS
_EPAD = _NW * _EPW
_NR = 10112
_RPT = _NR // _NSUB
_OPT = _N // _NSUB
_DW = 16

_PREC = jax.lax.Precision.HIGHEST


def _encode(x, w):
    def body(x_ref, w_ref, h_ref):
        h_ref[:] = jax.lax.dot_general(
            x_ref[:], w_ref[:], (((1,), (0,)), ((), ())),
            preferred_element_type=jnp.float32, precision=_PREC)

    return pl.pallas_call(
        body,
        out_shape=jax.ShapeDtypeStruct((_N, _D), jnp.float32),
    )(x, w)


def _sc_agg(h, srcr, dstr, z128, ones_k):
    mesh = plsc.VectorSubcoreMesh(core_axis_name="c", subcore_axis_name="s")

    @functools.partial(
        pl.kernel,
        out_type=(jax.ShapeDtypeStruct((_NCORE, _NR, _D), jnp.float32),
                  jax.ShapeDtypeStruct((_NCORE, _NR, _D), jnp.float32)),
        mesh=mesh,
        scratch_types=[
            pltpu.VMEM((_K,), jnp.int32),
            pltpu.VMEM((_K,), jnp.int32),
            pltpu.VMEM((_K, _D), jnp.float32),
            pltpu.VMEM((_K, _D), jnp.float32),
            pltpu.VMEM_SHARED((_NR, _D), jnp.float32),
        ],
    )
    def k(h_hbm, src_hbm, dst_hbm, z128_hbm, ones_hbm,
          agg_out, deg_out, sbuf, dbuf, gbuf, obuf, agg_sh):
        cid = lax.axis_index("c")
        sid = lax.axis_index("s")
        wid = cid * _NSUB + sid

        pltpu.sync_copy(z128_hbm, agg_sh.at[pl.ds(sid * _RPT, _RPT)])
        pltpu.sync_copy(ones_hbm, obuf)
        plsc.subcore_barrier()

        def step(j, carry):
            pltpu.sync_copy(src_hbm.at[wid, j], sbuf)
            pltpu.sync_copy(dst_hbm.at[wid, j], dbuf)
            pltpu.sync_copy(h_hbm.at[sbuf], gbuf)
            pltpu.sync_copy(gbuf, agg_sh.at[dbuf], add=True)
            return carry

        lax.fori_loop(0, _CHUNKS, step, 0)
        plsc.subcore_barrier()

        pltpu.sync_copy(agg_sh.at[pl.ds(sid * _RPT, _RPT)],
                        agg_out.at[cid, pl.ds(sid * _RPT, _RPT)])
        pltpu.sync_copy(z128_hbm, agg_sh.at[pl.ds(sid * _RPT, _RPT)])
        plsc.subcore_barrier()

        def step2(j, carry):
            pltpu.sync_copy(dst_hbm.at[wid, j], dbuf)
            pltpu.sync_copy(obuf, agg_sh.at[dbuf], add=True)
            return carry

        lax.fori_loop(0, _CHUNKS, step2, 0)
        plsc.subcore_barrier()

        pltpu.sync_copy(agg_sh.at[pl.ds(sid * _RPT, _RPT)],
                        deg_out.at[cid, pl.ds(sid * _RPT, _RPT)])

    return k(h, srcr, dstr, z128, ones_k)


def _head(p0, p1, d0, d1, batch2, be, w1, b1r, w2, b2r):
    def body(p0_ref, p1_ref, d0_ref, d1_ref, bt_ref, be_ref,
             w1_ref, b1_ref, w2_ref, b2_ref, out_ref):
        deg = jnp.maximum(d0_ref[:] + d1_ref[:], 1.0)
        z = jnp.maximum((p0_ref[:] + p1_ref[:]) / deg + be_ref[:], 0.0)
        gids = jax.lax.broadcasted_iota(jnp.int32, (1, _G), 1)
        m = (bt_ref[:] == gids).astype(jnp.float32)
        sums = jax.lax.dot_general(
            m, z, (((0,), (0,)), ((), ())),
            preferred_element_type=jnp.float32, precision=_PREC)
        ones_col = jnp.ones((_N, 1), jnp.float32)
        cnt = jax.lax.dot_general(
            m, ones_col, (((0,), (0,)), ((), ())),
            preferred_element_type=jnp.float32, precision=_PREC)
        pooled = sums / jnp.maximum(cnt, 1.0)
        hid = jnp.maximum(
            jax.lax.dot_general(pooled, w1_ref[:], (((1,), (0,)), ((), ())),
                                preferred_element_type=jnp.float32,
                                precision=_PREC) + b1_ref[:], 0.0)
        out_ref[:] = jax.lax.dot_general(
            hid, w2_ref[:], (((1,), (0,)), ((), ())),
            preferred_element_type=jnp.float32, precision=_PREC) + b2_ref[:]

    return pl.pallas_call(
        body,
        out_shape=jax.ShapeDtypeStruct((_G, _C), jnp.float32),
    )(p0, p1, d0, d1, batch2, be, w1, b1r, w2, b2r)


def kernel(x, edge_index, batch, W_enc, b_enc, W1, b1, W2, b2):
    h = _encode(x, W_enc)

    pad = _EPAD - _E
    srcp = jnp.concatenate(
        [edge_index[0], jnp.zeros((pad,), jnp.int32)]).reshape(_NW, _CHUNKS, _K)
    dstp = jnp.concatenate(
        [edge_index[1], jnp.full((pad,), _N, jnp.int32)]).reshape(_NW, _CHUNKS, _K)
    z128 = jnp.zeros((_RPT, _D), jnp.float32)
    ones_k = jnp.ones((_K, _D), jnp.float32)

    aggp, degp = _sc_agg(h, srcp, dstp, z128, ones_k)

    return _head(aggp[0, :_N], aggp[1, :_N],
                 degp[0, :_N, :1], degp[1, :_N, :1],
                 batch.reshape(_N, 1), b_enc.reshape(1, _D),
                 W1, b1.reshape(1, _H), W2, b2.reshape(1, _C))


# speedup vs baseline: 3.5259x; 1.1492x over previous
import functools

import jax
import jax.numpy as jnp
from jax import lax
from jax.experimental import pallas as pl
from jax.experimental.pallas import tpu as pltpu
from jax.experimental.pallas import tpu_sc as plsc

_N, _E, _D, _H, _C, _G = 10000, 320000, 128, 256, 10, 64
_NCORE, _NSUB = 2, 16
_NW = _NCORE * _NSUB
_K = 128
_CHUNKS = 80
_EPW = _K * _CHUNKS
_EPAD = _NW * _EPW
_NR = 10112
_RPT = _NR // _NSUB
_OPT = _N // _NSUB
_DW = 16

_PREC = jax.lax.Precision.HIGHEST


def _encode(x, w):
    def body(x_ref, w_ref, h_ref):
        h_ref[:] = jax.lax.dot_general(
            x_ref[:], w_ref[:], (((1,), (0,)), ((), ())),
            preferred_element_type=jnp.float32, precision=_PREC)

    return pl.pallas_call(
        body,
        out_shape=jax.ShapeDtypeStruct((_N, _D), jnp.float32),
    )(x, w)


def _sc_agg(h, srcr, dstr, z128, ones_k):
    mesh = plsc.VectorSubcoreMesh(core_axis_name="c", subcore_axis_name="s")

    @functools.partial(
        pl.kernel,
        out_type=(jax.ShapeDtypeStruct((_NCORE, _NR, _D), jnp.float32),
                  jax.ShapeDtypeStruct((_NCORE, _NR, _D), jnp.float32)),
        mesh=mesh,
        scratch_types=[
            pltpu.VMEM((_EPW,), jnp.int32),
            pltpu.VMEM((_EPW,), jnp.int32),
            pltpu.VMEM((_K, _D), jnp.float32),
            pltpu.VMEM_SHARED((_NR, _D), jnp.float32),
        ],
    )
    def k(h_hbm, src_hbm, dst_hbm, z128_hbm, ones_hbm,
          agg_out, deg_out, sbuf, dbuf, gbuf, agg_sh):
        cid = lax.axis_index("c")
        sid = lax.axis_index("s")
        wid = cid * _NSUB + sid

        pltpu.sync_copy(z128_hbm, agg_sh.at[pl.ds(sid * _RPT, _RPT)])
        pltpu.sync_copy(src_hbm.at[wid], sbuf)
        pltpu.sync_copy(dst_hbm.at[wid], dbuf)
        plsc.subcore_barrier()

        def step(j, carry):
            pltpu.sync_copy(h_hbm.at[sbuf.at[pl.ds(j * _K, _K)]], gbuf)
            pltpu.sync_copy(gbuf, agg_sh.at[dbuf.at[pl.ds(j * _K, _K)]],
                            add=True)
            return carry

        lax.fori_loop(0, _CHUNKS, step, 0)
        plsc.subcore_barrier()

        pltpu.sync_copy(agg_sh.at[pl.ds(sid * _RPT, _RPT)],
                        agg_out.at[cid, pl.ds(sid * _RPT, _RPT)])
        pltpu.sync_copy(z128_hbm, agg_sh.at[pl.ds(sid * _RPT, _RPT)])
        pltpu.sync_copy(ones_hbm, gbuf)
        plsc.subcore_barrier()

        def step2(j, carry):
            pltpu.sync_copy(gbuf, agg_sh.at[dbuf.at[pl.ds(j * _K, _K)]],
                            add=True)
            return carry

        lax.fori_loop(0, _CHUNKS, step2, 0)
        plsc.subcore_barrier()

        pltpu.sync_copy(agg_sh.at[pl.ds(sid * _RPT, _RPT)],
                        deg_out.at[cid, pl.ds(sid * _RPT, _RPT)])

    return k(h, srcr, dstr, z128, ones_k)


def _head(p0, p1, d0, d1, batch2, be, w1, b1r, w2, b2r):
    def body(p0_ref, p1_ref, d0_ref, d1_ref, bt_ref, be_ref,
             w1_ref, b1_ref, w2_ref, b2_ref, out_ref):
        deg = jnp.maximum(d0_ref[:] + d1_ref[:], 1.0)
        z = jnp.maximum((p0_ref[:] + p1_ref[:]) / deg + be_ref[:], 0.0)
        gids = jax.lax.broadcasted_iota(jnp.int32, (1, _G), 1)
        m = (bt_ref[:] == gids).astype(jnp.float32)
        sums = jax.lax.dot_general(
            m, z, (((0,), (0,)), ((), ())),
            preferred_element_type=jnp.float32, precision=_PREC)
        ones_col = jnp.ones((_N, 1), jnp.float32)
        cnt = jax.lax.dot_general(
            m, ones_col, (((0,), (0,)), ((), ())),
            preferred_element_type=jnp.float32, precision=_PREC)
        pooled = sums / jnp.maximum(cnt, 1.0)
        hid = jnp.maximum(
            jax.lax.dot_general(pooled, w1_ref[:], (((1,), (0,)), ((), ())),
                                preferred_element_type=jnp.float32,
                                precision=_PREC) + b1_ref[:], 0.0)
        out_ref[:] = jax.lax.dot_general(
            hid, w2_ref[:], (((1,), (0,)), ((), ())),
            preferred_element_type=jnp.float32, precision=_PREC) + b2_ref[:]

    return pl.pallas_call(
        body,
        out_shape=jax.ShapeDtypeStruct((_G, _C), jnp.float32),
    )(p0, p1, d0, d1, batch2, be, w1, b1r, w2, b2r)


def kernel(x, edge_index, batch, W_enc, b_enc, W1, b1, W2, b2):
    h = _encode(x, W_enc)

    pad = _EPAD - _E
    srcp = jnp.concatenate(
        [edge_index[0], jnp.zeros((pad,), jnp.int32)]).reshape(_NW, _EPW)
    dstp = jnp.concatenate(
        [edge_index[1], jnp.full((pad,), _N, jnp.int32)]).reshape(_NW, _EPW)
    z128 = jnp.zeros((_RPT, _D), jnp.float32)
    ones_k = jnp.ones((_K, _D), jnp.float32)

    aggp, degp = _sc_agg(h, srcp, dstp, z128, ones_k)

    return _head(aggp[0, :_N], aggp[1, :_N],
                 degp[0, :_N, :1], degp[1, :_N, :1],
                 batch.reshape(_N, 1), b_enc.reshape(1, _D),
                 W1, b1.reshape(1, _H), W2, b2.reshape(1, _C))


# speedup vs baseline: 4.6054x; 1.3062x over previous
import functools

import jax
import jax.numpy as jnp
from jax import lax
from jax.experimental import pallas as pl
from jax.experimental.pallas import tpu as pltpu
from jax.experimental.pallas import tpu_sc as plsc

_N, _E, _D, _H, _C, _G = 10000, 320000, 128, 256, 10, 64
_NCORE, _NSUB = 2, 16
_NW = _NCORE * _NSUB
_K = 128
_CHUNKS = 80
_EPW = _K * _CHUNKS
_EPAD = _NW * _EPW
_NR = 10112
_RPT = _NR // _NSUB
_OPT = _N // _NSUB
_DW = 16

_PREC = jax.lax.Precision.HIGHEST


def _encode(x, w):
    def body(x_ref, w_ref, h_ref):
        h_ref[:] = jax.lax.dot_general(
            x_ref[:], w_ref[:], (((1,), (0,)), ((), ())),
            preferred_element_type=jnp.float32, precision=_PREC)

    return pl.pallas_call(
        body,
        out_shape=jax.ShapeDtypeStruct((_N, _D), jnp.float32),
    )(x, w)


def _sc_agg(h, srcr, dstr, z128):
    mesh = plsc.VectorSubcoreMesh(core_axis_name="c", subcore_axis_name="s")

    @functools.partial(
        pl.kernel,
        out_type=jax.ShapeDtypeStruct((_NCORE, _NR, _D), jnp.float32),
        mesh=mesh,
        scratch_types=[
            pltpu.VMEM((_EPW,), jnp.int32),
            pltpu.VMEM((_EPW,), jnp.int32),
            pltpu.VMEM((_K, _D), jnp.float32),
            pltpu.VMEM_SHARED((_NR, _D), jnp.float32),
        ],
    )
    def k(h_hbm, src_hbm, dst_hbm, z128_hbm,
          agg_out, sbuf, dbuf, gbuf, agg_sh):
        cid = lax.axis_index("c")
        sid = lax.axis_index("s")
        wid = cid * _NSUB + sid

        pltpu.sync_copy(z128_hbm, agg_sh.at[pl.ds(sid * _RPT, _RPT)])
        pltpu.sync_copy(src_hbm.at[wid], sbuf)
        pltpu.sync_copy(dst_hbm.at[wid], dbuf)
        plsc.subcore_barrier()

        def step(j, carry):
            pltpu.sync_copy(h_hbm.at[sbuf.at[pl.ds(j * _K, _K)]], gbuf)
            pltpu.sync_copy(gbuf, agg_sh.at[dbuf.at[pl.ds(j * _K, _K)]],
                            add=True)
            return carry

        lax.fori_loop(0, _CHUNKS, step, 0)
        plsc.subcore_barrier()

        pltpu.sync_copy(agg_sh.at[pl.ds(sid * _RPT, _RPT)],
                        agg_out.at[cid, pl.ds(sid * _RPT, _RPT)])

    return k(h, srcr, dstr, z128)


_M = 2048
_DCH = 160


def _deg_tc(d2):
    def body(d_ref, out_ref):
        isub = jax.lax.broadcasted_iota(jnp.int32, (_D, 1), 0)

        def outer(b, acc):
            blk = d_ref[pl.ds(b * 8, 8), :]
            for s in range(8):
                row = blk[s:s + 1, :]
                ohq = ((row >> 7) == isub).astype(jnp.float32)
                ohr = ((row & 127) == isub).astype(jnp.float32)
                acc = acc + jax.lax.dot_general(
                    ohq, ohr, (((1,), (1,)), ((), ())),
                    preferred_element_type=jnp.float32, precision=_PREC)
            return acc

        out_ref[:] = lax.fori_loop(
            0, _DCH // 8, outer, jnp.zeros((_D, _D), jnp.float32))

    return pl.pallas_call(
        body,
        out_shape=jax.ShapeDtypeStruct((_D, _D), jnp.float32),
    )(d2)


def _head(p0, p1, dd, batch2, be, w1, b1r, w2, b2r):
    def body(p0_ref, p1_ref, dd_ref, bt_ref, be_ref,
             w1_ref, b1_ref, w2_ref, b2_ref, out_ref):
        deg = jnp.maximum(dd_ref[:], 1.0)
        z = jnp.maximum((p0_ref[:] + p1_ref[:]) / deg + be_ref[:], 0.0)
        gids = jax.lax.broadcasted_iota(jnp.int32, (1, _G), 1)
        m = (bt_ref[:] == gids).astype(jnp.float32)
        sums = jax.lax.dot_general(
            m, z, (((0,), (0,)), ((), ())),
            preferred_element_type=jnp.float32, precision=_PREC)
        ones_col = jnp.ones((_N, 1), jnp.float32)
        cnt = jax.lax.dot_general(
            m, ones_col, (((0,), (0,)), ((), ())),
            preferred_element_type=jnp.float32, precision=_PREC)
        pooled = sums / jnp.maximum(cnt, 1.0)
        hid = jnp.maximum(
            jax.lax.dot_general(pooled, w1_ref[:], (((1,), (0,)), ((), ())),
                                preferred_element_type=jnp.float32,
                                precision=_PREC) + b1_ref[:], 0.0)
        out_ref[:] = jax.lax.dot_general(
            hid, w2_ref[:], (((1,), (0,)), ((), ())),
            preferred_element_type=jnp.float32, precision=_PREC) + b2_ref[:]

    return pl.pallas_call(
        body,
        out_shape=jax.ShapeDtypeStruct((_G, _C), jnp.float32),
    )(p0, p1, dd, batch2, be, w1, b1r, w2, b2r)


def kernel(x, edge_index, batch, W_enc, b_enc, W1, b1, W2, b2):
    h = _encode(x, W_enc)

    pad = _EPAD - _E
    srcp = jnp.concatenate(
        [edge_index[0], jnp.zeros((pad,), jnp.int32)]).reshape(_NW, _EPW)
    dstp = jnp.concatenate(
        [edge_index[1], jnp.full((pad,), _N, jnp.int32)]).reshape(_NW, _EPW)
    z128 = jnp.zeros((_RPT, _D), jnp.float32)

    aggp = _sc_agg(h, srcp, dstp, z128)

    deg_grid = _deg_tc(dstp.reshape(_DCH, _M))
    deg = deg_grid.reshape(_D * _D, 1)[:_N]

    return _head(aggp[0, :_N], aggp[1, :_N], deg,
                 batch.reshape(_N, 1), b_enc.reshape(1, _D),
                 W1, b1.reshape(1, _H), W2, b2.reshape(1, _C))


# speedup vs baseline: 5.2135x; 1.1320x over previous
import functools

import jax
import jax.numpy as jnp
from jax import lax
from jax.experimental import pallas as pl
from jax.experimental.pallas import tpu as pltpu
from jax.experimental.pallas import tpu_sc as plsc

_N, _E, _D, _H, _C, _G = 10000, 320000, 128, 256, 10, 64
_NCORE, _NSUB = 2, 16
_NW = _NCORE * _NSUB
_K = 64
_CHUNKS = 160
_EPW = _K * _CHUNKS
_EPAD = _NW * _EPW
_NR = 10112
_RPT = _NR // _NSUB
_OPT = _N // _NSUB
_DW = 16

_PREC = jax.lax.Precision.HIGHEST


def _encode(x, w):
    def body(x_ref, w_ref, h_ref):
        h_ref[:] = jax.lax.dot_general(
            x_ref[:], w_ref[:], (((1,), (0,)), ((), ())),
            preferred_element_type=jnp.float32, precision=_PREC)

    return pl.pallas_call(
        body,
        out_shape=jax.ShapeDtypeStruct((_N, _D), jnp.float32),
    )(x, w)


def _sc_agg(h, srcr, dstr, z128):
    mesh = plsc.VectorSubcoreMesh(core_axis_name="c", subcore_axis_name="s")

    @functools.partial(
        pl.kernel,
        out_type=jax.ShapeDtypeStruct((_NCORE, _NR, _D), jnp.float32),
        mesh=mesh,
        scratch_types=[
            pltpu.VMEM((_EPW,), jnp.int32),
            pltpu.VMEM((_EPW,), jnp.int32),
            pltpu.VMEM((_K, _D), jnp.float32),
            pltpu.VMEM((_K, _D), jnp.float32),
            pltpu.SemaphoreType.DMA,
            pltpu.SemaphoreType.DMA,
            pltpu.VMEM_SHARED((_NR, _D), jnp.float32),
        ],
    )
    def k(h_hbm, src_hbm, dst_hbm, z128_hbm,
          agg_out, sbuf, dbuf, gbuf0, gbuf1, sem0, sem1, agg_sh):
        cid = lax.axis_index("c")
        sid = lax.axis_index("s")
        wid = cid * _NSUB + sid

        pltpu.sync_copy(z128_hbm, agg_sh.at[pl.ds(sid * _RPT, _RPT)])
        pltpu.sync_copy(src_hbm.at[wid], sbuf)
        pltpu.sync_copy(dst_hbm.at[wid], dbuf)
        plsc.subcore_barrier()

        pltpu.async_copy(h_hbm.at[sbuf.at[pl.ds(0, _K)]], gbuf0, sem0)

        def step(t, carry):
            o0 = 2 * t * _K
            o1 = o0 + _K
            pltpu.async_copy(h_hbm.at[sbuf.at[pl.ds(o1, _K)]], gbuf1, sem1)
            pltpu.make_async_copy(
                h_hbm.at[sbuf.at[pl.ds(o0, _K)]], gbuf0, sem0).wait()
            pltpu.sync_copy(gbuf0, agg_sh.at[dbuf.at[pl.ds(o0, _K)]],
                            add=True)

            @pl.when(t != _CHUNKS // 2 - 1)
            def _():
                pltpu.async_copy(
                    h_hbm.at[sbuf.at[pl.ds(o1 + _K, _K)]], gbuf0, sem0)

            pltpu.make_async_copy(
                h_hbm.at[sbuf.at[pl.ds(o1, _K)]], gbuf1, sem1).wait()
            pltpu.sync_copy(gbuf1, agg_sh.at[dbuf.at[pl.ds(o1, _K)]],
                            add=True)
            return carry

        lax.fori_loop(0, _CHUNKS // 2, step, 0)
        plsc.subcore_barrier()

        pltpu.sync_copy(agg_sh.at[pl.ds(sid * _RPT, _RPT)],
                        agg_out.at[cid, pl.ds(sid * _RPT, _RPT)])

    return k(h, srcr, dstr, z128)


_M = 2048
_DCH = 160


def _deg_tc(d2):
    def body(d_ref, out_ref):
        isub = jax.lax.broadcasted_iota(jnp.int32, (_D, 1), 0)

        def outer(b, acc):
            blk = d_ref[pl.ds(b * 8, 8), :]
            for s in range(8):
                row = blk[s:s + 1, :]
                ohq = ((row >> 7) == isub).astype(jnp.float32)
                ohr = ((row & 127) == isub).astype(jnp.float32)
                acc = acc + jax.lax.dot_general(
                    ohq, ohr, (((1,), (1,)), ((), ())),
                    preferred_element_type=jnp.float32, precision=_PREC)
            return acc

        out_ref[:] = lax.fori_loop(
            0, _DCH // 8, outer, jnp.zeros((_D, _D), jnp.float32))

    return pl.pallas_call(
        body,
        out_shape=jax.ShapeDtypeStruct((_D, _D), jnp.float32),
    )(d2)


def _head(p0, p1, dd, batch2, be, w1, b1r, w2, b2r):
    def body(p0_ref, p1_ref, dd_ref, bt_ref, be_ref,
             w1_ref, b1_ref, w2_ref, b2_ref, out_ref):
        deg = jnp.maximum(dd_ref[:], 1.0)
        z = jnp.maximum((p0_ref[:] + p1_ref[:]) / deg + be_ref[:], 0.0)
        gids = jax.lax.broadcasted_iota(jnp.int32, (1, _G), 1)
        m = (bt_ref[:] == gids).astype(jnp.float32)
        sums = jax.lax.dot_general(
            m, z, (((0,), (0,)), ((), ())),
            preferred_element_type=jnp.float32, precision=_PREC)
        ones_col = jnp.ones((_N, 1), jnp.float32)
        cnt = jax.lax.dot_general(
            m, ones_col, (((0,), (0,)), ((), ())),
            preferred_element_type=jnp.float32, precision=_PREC)
        pooled = sums / jnp.maximum(cnt, 1.0)
        hid = jnp.maximum(
            jax.lax.dot_general(pooled, w1_ref[:], (((1,), (0,)), ((), ())),
                                preferred_element_type=jnp.float32,
                                precision=_PREC) + b1_ref[:], 0.0)
        out_ref[:] = jax.lax.dot_general(
            hid, w2_ref[:], (((1,), (0,)), ((), ())),
            preferred_element_type=jnp.float32, precision=_PREC) + b2_ref[:]

    return pl.pallas_call(
        body,
        out_shape=jax.ShapeDtypeStruct((_G, _C), jnp.float32),
    )(p0, p1, dd, batch2, be, w1, b1r, w2, b2r)


def kernel(x, edge_index, batch, W_enc, b_enc, W1, b1, W2, b2):
    h = _encode(x, W_enc)

    pad = _EPAD - _E
    srcp = jnp.concatenate(
        [edge_index[0], jnp.zeros((pad,), jnp.int32)]).reshape(_NW, _EPW)
    dstp = jnp.concatenate(
        [edge_index[1], jnp.full((pad,), _N, jnp.int32)]).reshape(_NW, _EPW)
    z128 = jnp.zeros((_RPT, _D), jnp.float32)

    aggp = _sc_agg(h, srcp, dstp, z128)

    deg_grid = _deg_tc(dstp.reshape(_DCH, _M))
    deg = deg_grid.reshape(_D * _D, 1)[:_N]

    return _head(aggp[0, :_N], aggp[1, :_N], deg,
                 batch.reshape(_N, 1), b_enc.reshape(1, _D),
                 W1, b1.reshape(1, _H), W2, b2.reshape(1, _C))
